# Initial kernel scaffold; baseline (speedup 1.0000x reference)
import jax
import jax.numpy as jnp
from jax.experimental import pallas as pl


def kernel(features, targets, centroids):
    raise NotImplementedError("write your pallas kernel here")


# speedup vs baseline: 3.0092x; 3.0092x over previous
import jax
import jax.numpy as jnp
from jax import lax
from jax.experimental import pallas as pl
from jax.experimental.pallas import tpu as pltpu
from jax.experimental.pallas import tpu_sc as plsc

_C = 100
_CP = 128
_D = 512
_B = 4096
_MARGIN = 2.0
_L = 16
_NC = 2
_NS = 16
_NW = _NC * _NS
_RPW = _B // _NW
_CHUNK = 64
_NCHUNK = _RPW // _CHUNK


def _sc_body(features_hbm, targets_hbm, out_acc, out_cnt, out_sq,
             idx_v, rows_v, acc_v, cnt_v, sq_v):
    cid = lax.axis_index("c")
    sid = lax.axis_index("s")
    wid = cid * _NS + sid
    rbase = wid * _RPW

    zero = jnp.zeros((_L,), jnp.float32)
    ones = jnp.ones((_L,), jnp.float32)
    lane = lax.iota(jnp.int32, _L)

    def zbody(r, carry):
        for k in range(_D // _L):
            acc_v[r, pl.ds(k * _L, _L)] = zero
        cnt_v[r] = zero
        return carry

    lax.fori_loop(0, _CP, zbody, 0)

    pltpu.sync_copy(targets_hbm.at[pl.ds(rbase, _RPW)], idx_v)

    sq = zero
    for c in range(_NCHUNK):
        pltpu.sync_copy(
            features_hbm.at[pl.ds(rbase + c * _CHUNK, _CHUNK)], rows_v)

        def rbody(r, s, c=c):
            t = plsc.load_gather(idx_v, [jnp.full((_L,), c * _CHUNK, jnp.int32) + r])
            plsc.addupdate_scatter(cnt_v, [t, lane], ones)
            for k in range(_D // _L):
                v = rows_v[r, pl.ds(k * _L, _L)]
                plsc.addupdate_scatter(acc_v, [t, lane + (k * _L)], v)
                s = s + v * v
            return s

        sq = lax.fori_loop(0, _CHUNK, rbody, sq)

    sq_v[...] = sq
    pltpu.sync_copy(sq_v, out_sq.at[wid])
    pltpu.sync_copy(acc_v, out_acc.at[pl.ds(wid * _CP, _CP)])
    pltpu.sync_copy(cnt_v, out_cnt.at[pl.ds(wid * _CP, _CP)])


_sc_call = pl.kernel(
    _sc_body,
    out_type=(
        jax.ShapeDtypeStruct((_NW * _CP, _D), jnp.float32),
        jax.ShapeDtypeStruct((_NW * _CP, _L), jnp.float32),
        jax.ShapeDtypeStruct((_NW, _L), jnp.float32),
    ),
    mesh=plsc.VectorSubcoreMesh(core_axis_name="c", subcore_axis_name="s"),
    compiler_params=pltpu.CompilerParams(needs_layout_passes=False),
    scratch_types=(
        pltpu.VMEM((_RPW,), jnp.int32),
        pltpu.VMEM((_CHUNK, _D), jnp.float32),
        pltpu.VMEM((_CP, _D), jnp.float32),
        pltpu.VMEM((_CP, _L), jnp.float32),
        pltpu.VMEM((_L,), jnp.float32),
    ),
)


def _tc_body(acc_ref, cnt_ref, sq_ref, out_ref):
    sums = acc_ref[0:_CP, :]
    cnts = cnt_ref[0:_CP, :]
    for w in range(1, _NW):
        sums = sums + acc_ref[w * _CP:(w + 1) * _CP, :]
        cnts = cnts + cnt_ref[w * _CP:(w + 1) * _CP, :]
    counts = cnts[:, 0:1]
    sumsq = jnp.sum(sq_ref[...])
    centers = sums / jnp.maximum(counts, 1.0)
    norms = jnp.sum(centers * centers, axis=1, keepdims=True)
    intra = (sumsq - jnp.sum(counts * norms)) / _B

    g = lax.dot_general(centers, centers, (((1,), (1,)), ((), ())),
                        preferred_element_type=jnp.float32)
    ones_col = jnp.ones((_CP, 1), jnp.float32)
    nj = lax.dot_general(ones_col, norms, (((1,), (1,)), ((), ())))
    d2 = norms + nj - 2.0 * g
    hinge = jnp.maximum(_MARGIN - d2, 0.0)
    ri = lax.broadcasted_iota(jnp.int32, (_CP, _CP), 0)
    cj = lax.broadcasted_iota(jnp.int32, (_CP, _CP), 1)
    valid = (ri != cj) & (ri < _C) & (cj < _C)
    hs = (jnp.sum(jnp.where(valid, hinge, 0.0)) * 0.5
          + jnp.sum(jnp.where((ri == 1) & (cj == 2), hinge, 0.0)))
    n_pairs = _C * (_C - 1) // 2
    out_ref[...] = jnp.broadcast_to(intra + hs / n_pairs, (1, 1))


_tc_call = pl.pallas_call(
    _tc_body,
    out_shape=jax.ShapeDtypeStruct((1, 1), jnp.float32),
)


@jax.jit
def kernel(features, targets, centroids):
    del centroids
    acc, cnt, sq = _sc_call(features, targets)
    loss = _tc_call(acc, cnt, sq)
    return jnp.reshape(loss, ())
